# baseline (speedup 1.0000x reference)
import functools

import jax
import jax.numpy as jnp
from jax import lax
from jax.experimental import pallas as pl
from jax.experimental.pallas import tpu as pltpu
from jax.experimental.pallas import tpu_sc as plsc

_H = 224
_W = 224
_PAD = 16
_MAXR = 4
_HP = _H + 2 * _PAD
_WP = _W + 2 * _PAD
_NP = 128
_NG = _NP // 16
_YU = 56
_SLAB = _YU + 2 * _PAD
_OCH = 4


def _boxmean_body(x_ref, out_ref):
    img = x_ref[0, 0]
    top = img[0:1, :]
    bot = img[_H - 1:_H, :]
    pimg = jnp.concatenate([top] * _MAXR + [img] + [bot] * _MAXR, axis=0)
    left = pimg[:, 0:1]
    right = pimg[:, _W - 1:_W]
    pimg = jnp.concatenate([left] * _MAXR + [pimg] + [right] * _MAXR,
                           axis=1)
    for r in range(1, _MAXR + 1):
        rs = pimg[_MAXR - r:_MAXR - r + _H, :]
        for k in range(-r + 1, r + 1):
            rs = rs + pimg[_MAXR + k:_MAXR + k + _H, :]
        bs = rs[:, _MAXR - r:_MAXR - r + _W]
        for k in range(-r + 1, r + 1):
            bs = bs + rs[:, _MAXR + k:_MAXR + k + _W]
        m = bs * (1.0 / float((2 * r + 1) ** 2))
        base = (r - 1) * _HP
        rowpad = jnp.concatenate(
            [jnp.broadcast_to(m[0:1, :], (_PAD, _W)), m,
             jnp.broadcast_to(m[_H - 1:_H, :], (_PAD, _W))], axis=0)
        out_ref[base:base + _HP, _PAD:_PAD + _W] = rowpad
        lcol = out_ref[base:base + _HP, _PAD:_PAD + 1]
        out_ref[base:base + _HP, 0:_PAD] = jnp.broadcast_to(lcol,
                                                            (_HP, _PAD))
        rcol = out_ref[base:base + _HP, _PAD + _W - 1:_PAD + _W]
        out_ref[base:base + _HP, _PAD + _W:_WP] = jnp.broadcast_to(
            rcol, (_HP, _PAD))


def _boxmean_rows(x):
    B = x.shape[0]
    return pl.pallas_call(
        _boxmean_body,
        grid=(B,),
        in_specs=[pl.BlockSpec((1, 1, _H, _W), lambda b: (b, 0, 0, 0))],
        out_specs=pl.BlockSpec((_MAXR * _HP, _WP), lambda b: (b, 0)),
        out_shape=jax.ShapeDtypeStruct((B * _MAXR * _HP, _WP), jnp.float32),
    )(x)


def _sc_sample_body(mrows, params, thrt, out,
                    par_v, thr_v, slabs, oba, obb, ssem, oa, ob):
    wid = lax.axis_index("s") * 2 + lax.axis_index("c")
    b = wid // 16
    rem = wid - b * 16
    g = rem // 2
    half = rem - g * 2
    obufs = (oba, obb)
    osems = (oa, ob)

    pltpu.sync_copy(params.at[g], par_v)
    pltpu.sync_copy(thrt.at[g], thr_v)
    par_sy1 = par_v[0]
    par_sy2 = par_v[1]
    par_sx1 = par_v[2]
    par_sx2 = par_v[3]
    ridxv = par_v[4]
    thrv = thr_v[...]
    fbase1 = (ridxv * _SLAB + par_sy1) * _WP + par_sx1
    fbase2 = (ridxv * _SLAB + par_sy2) * _WP + par_sx2

    def unit_body(u, _):
        y0 = half * (2 * _YU) + u * _YU

        def slab_copy(rr):
            return pltpu.make_async_copy(
                mrows.at[pl.ds(((b * _MAXR + rr) * _HP + y0) * _WP,
                               _SLAB * _WP)],
                slabs.at[pl.ds(rr * _SLAB * _WP, _SLAB * _WP)], ssem)

        for rr in range(_MAXR):
            slab_copy(rr).start()
        for rr in range(_MAXR):
            slab_copy(rr).wait()

        def out_copy(c, pi):
            return pltpu.make_async_copy(
                obufs[pi],
                out.at[b, pl.ds(y0 + c * _OCH, _OCH), :,
                       pl.ds(g * 16, 16)],
                osems[pi])

        for c in range(_YU // _OCH):
            pi = c % 2
            if c >= 2:
                out_copy(c - 2, pi).wait()
            ob_ref = obufs[pi]
            for yy in range(_OCH):
                y = c * _OCH + yy
                ybase1 = fbase1 + y * _WP
                ybase2 = fbase2 + y * _WP

                @plsc.parallel_loop(0, _W, unroll=8)
                def _cols(x):
                    xv = jnp.full((16,), x, jnp.int32)
                    v1 = plsc.load_gather(slabs, [ybase1 + xv])
                    v2 = plsc.load_gather(slabs, [ybase2 + xv])
                    ob_ref[yy, x, :] = v1 - v2 - thrv

            out_copy(c, pi).start()
        nch = _YU // _OCH
        out_copy(nch - 2, (nch - 2) % 2).wait()
        out_copy(nch - 1, (nch - 1) % 2).wait()
        return 0

    lax.fori_loop(0, 2, unit_body, 0)


def _sc_sample(mrows, params, thrt, B):
    mesh = plsc.VectorSubcoreMesh(core_axis_name="c", subcore_axis_name="s")
    f = functools.partial(
        pl.kernel,
        mesh=mesh,
        out_type=jax.ShapeDtypeStruct((B, _H, _W, _NP), jnp.float32),
        scratch_types=[
            pltpu.VMEM((5, 16), jnp.int32),
            pltpu.VMEM((16,), jnp.float32),
            pltpu.VMEM((_MAXR * _SLAB * _WP,), jnp.float32),
            pltpu.VMEM((_OCH, _W, 16), jnp.float32),
            pltpu.VMEM((_OCH, _W, 16), jnp.float32),
            pltpu.SemaphoreType.DMA,
            pltpu.SemaphoreType.DMA,
            pltpu.SemaphoreType.DMA,
        ],
        compiler_params=pltpu.CompilerParams(use_tc_tiling_on_sc=False,
                                             needs_layout_passes=False),
    )(_sc_sample_body)
    return f(mrows, params, thrt)


@jax.jit
def kernel(x, offset_x1, offset_x2, offset_y1, offset_y2, thresholds, radii):
    B = x.shape[0]
    mrows = _boxmean_rows(x)

    s1y = (jnp.floor(offset_y1) + _PAD).astype(jnp.int32)
    s1x = (jnp.floor(offset_x1) + _PAD).astype(jnp.int32)
    s2y = (jnp.floor(offset_y2) + _PAD).astype(jnp.int32)
    s2x = (jnp.floor(offset_x2) + _PAD).astype(jnp.int32)
    ridx = (radii - 1).astype(jnp.int32)

    params = jnp.stack([s1y, s2y, s1x, s2x, ridx],
                       axis=0).reshape(5, _NG, 16).transpose(1, 0, 2)
    thrt = thresholds.astype(jnp.float32).reshape(_NG, 16)

    out4 = _sc_sample(mrows.reshape(-1), params, thrt, B)
    return jnp.transpose(out4, (0, 3, 1, 2))


# speedup vs baseline: 1.0557x; 1.0557x over previous
import functools

import jax
import jax.numpy as jnp
from jax import lax
from jax.experimental import pallas as pl
from jax.experimental.pallas import tpu as pltpu
from jax.experimental.pallas import tpu_sc as plsc

_H = 224
_W = 224
_PAD = 16
_MAXR = 4
_HP = _H + 2 * _PAD
_WP = _W + 2 * _PAD
_NP = 128
_NG = _NP // 16
_YU = 56
_SLAB = _YU + 2 * _PAD
_OCH = 4


def _boxmean_body(x_ref, out_ref):
    img = x_ref[0, 0]
    top = img[0:1, :]
    bot = img[_H - 1:_H, :]
    pimg = jnp.concatenate([top] * _MAXR + [img] + [bot] * _MAXR, axis=0)
    left = pimg[:, 0:1]
    right = pimg[:, _W - 1:_W]
    pimg = jnp.concatenate([left] * _MAXR + [pimg] + [right] * _MAXR,
                           axis=1)
    for r in range(1, _MAXR + 1):
        rs = pimg[_MAXR - r:_MAXR - r + _H, :]
        for k in range(-r + 1, r + 1):
            rs = rs + pimg[_MAXR + k:_MAXR + k + _H, :]
        bs = rs[:, _MAXR - r:_MAXR - r + _W]
        for k in range(-r + 1, r + 1):
            bs = bs + rs[:, _MAXR + k:_MAXR + k + _W]
        m = bs * (1.0 / float((2 * r + 1) ** 2))
        base = (r - 1) * _HP
        rowpad = jnp.concatenate(
            [jnp.broadcast_to(m[0:1, :], (_PAD, _W)), m,
             jnp.broadcast_to(m[_H - 1:_H, :], (_PAD, _W))], axis=0)
        out_ref[base:base + _HP, _PAD:_PAD + _W] = rowpad
        lcol = out_ref[base:base + _HP, _PAD:_PAD + 1]
        out_ref[base:base + _HP, 0:_PAD] = jnp.broadcast_to(lcol,
                                                            (_HP, _PAD))
        rcol = out_ref[base:base + _HP, _PAD + _W - 1:_PAD + _W]
        out_ref[base:base + _HP, _PAD + _W:_WP] = jnp.broadcast_to(
            rcol, (_HP, _PAD))


def _boxmean_rows(x):
    B = x.shape[0]
    return pl.pallas_call(
        _boxmean_body,
        grid=(B,),
        in_specs=[pl.BlockSpec((1, 1, _H, _W), lambda b: (b, 0, 0, 0))],
        out_specs=pl.BlockSpec((_MAXR * _HP, _WP), lambda b: (b, 0)),
        out_shape=jax.ShapeDtypeStruct((B * _MAXR * _HP, _WP), jnp.float32),
    )(x)


def _sc_sample_body(mrows, params, thrt, out,
                    par_v, thr_v, slabs, oba, obb, ssem, oa, ob):
    wid = lax.axis_index("s") * 2 + lax.axis_index("c")
    b = wid // 16
    rem = wid - b * 16
    g = rem // 2
    half = rem - g * 2
    obufs = (oba, obb)
    osems = (oa, ob)

    pltpu.sync_copy(params.at[g], par_v)
    pltpu.sync_copy(thrt.at[g], thr_v)
    par_sy1 = par_v[0]
    par_sy2 = par_v[1]
    par_sx1 = par_v[2]
    par_sx2 = par_v[3]
    ridxv = par_v[4]
    thrv = thr_v[...]
    fbase1 = (ridxv * _SLAB + par_sy1) * _WP + par_sx1
    fbase2 = (ridxv * _SLAB + par_sy2) * _WP + par_sx2

    def unit_body(u, _):
        y0 = half * (2 * _YU) + u * _YU

        def slab_copy(rr):
            return pltpu.make_async_copy(
                mrows.at[pl.ds(((b * _MAXR + rr) * _HP + y0) * _WP,
                               _SLAB * _WP)],
                slabs.at[pl.ds(rr * _SLAB * _WP, _SLAB * _WP)], ssem)

        for rr in range(_MAXR):
            slab_copy(rr).start()
        for rr in range(_MAXR):
            slab_copy(rr).wait()

        def out_copy(c, pi):
            return pltpu.make_async_copy(
                obufs[pi],
                out.at[b, pl.ds(y0 + c * _OCH, _OCH), :,
                       pl.ds(g * 16, 16)],
                osems[pi])

        for c in range(_YU // _OCH):
            pi = c % 2
            if c >= 2:
                out_copy(c - 2, pi).wait()
            ob_ref = obufs[pi]
            for yy in range(_OCH):
                y = c * _OCH + yy
                ybase1 = fbase1 + y * _WP
                ybase2 = fbase2 + y * _WP

                @plsc.parallel_loop(0, _W, unroll=4,
                                    carry=(ybase1, ybase2))
                def _cols(x, idx):
                    i1, i2 = idx
                    v1 = plsc.load_gather(slabs, [i1])
                    v2 = plsc.load_gather(slabs, [i2])
                    ob_ref[yy, x, :] = v1 - v2 - thrv
                    return (i1 + 1, i2 + 1)

            out_copy(c, pi).start()
        nch = _YU // _OCH
        out_copy(nch - 2, (nch - 2) % 2).wait()
        out_copy(nch - 1, (nch - 1) % 2).wait()
        return 0

    lax.fori_loop(0, 2, unit_body, 0)


def _sc_sample(mrows, params, thrt, B):
    mesh = plsc.VectorSubcoreMesh(core_axis_name="c", subcore_axis_name="s")
    f = functools.partial(
        pl.kernel,
        mesh=mesh,
        out_type=jax.ShapeDtypeStruct((B, _H, _W, _NP), jnp.float32),
        scratch_types=[
            pltpu.VMEM((5, 16), jnp.int32),
            pltpu.VMEM((16,), jnp.float32),
            pltpu.VMEM((_MAXR * _SLAB * _WP,), jnp.float32),
            pltpu.VMEM((_OCH, _W, 16), jnp.float32),
            pltpu.VMEM((_OCH, _W, 16), jnp.float32),
            pltpu.SemaphoreType.DMA,
            pltpu.SemaphoreType.DMA,
            pltpu.SemaphoreType.DMA,
        ],
        compiler_params=pltpu.CompilerParams(use_tc_tiling_on_sc=False,
                                             needs_layout_passes=False),
    )(_sc_sample_body)
    return f(mrows, params, thrt)


@jax.jit
def kernel(x, offset_x1, offset_x2, offset_y1, offset_y2, thresholds, radii):
    B = x.shape[0]
    mrows = _boxmean_rows(x)

    s1y = (jnp.floor(offset_y1) + _PAD).astype(jnp.int32)
    s1x = (jnp.floor(offset_x1) + _PAD).astype(jnp.int32)
    s2y = (jnp.floor(offset_y2) + _PAD).astype(jnp.int32)
    s2x = (jnp.floor(offset_x2) + _PAD).astype(jnp.int32)
    ridx = (radii - 1).astype(jnp.int32)

    params = jnp.stack([s1y, s2y, s1x, s2x, ridx],
                       axis=0).reshape(5, _NG, 16).transpose(1, 0, 2)
    thrt = thresholds.astype(jnp.float32).reshape(_NG, 16)

    out4 = _sc_sample(mrows.reshape(-1), params, thrt, B)
    return jnp.transpose(out4, (0, 3, 1, 2))


# speedup vs baseline: 1.0857x; 1.0284x over previous
import functools

import jax
import jax.numpy as jnp
from jax import lax
from jax.experimental import pallas as pl
from jax.experimental.pallas import tpu as pltpu
from jax.experimental.pallas import tpu_sc as plsc

_H = 224
_W = 224
_PAD = 16
_MAXR = 4
_HP = _H + 2 * _PAD
_WP = _W + 2 * _PAD
_NP = 128
_NG = _NP // 16
_YU = 56
_SLAB = _YU + 2 * _PAD
_OCH = 4


def _boxmean_body(x_ref, out_ref):
    img = x_ref[0, 0]
    top = img[0:1, :]
    bot = img[_H - 1:_H, :]
    pimg = jnp.concatenate([top] * _MAXR + [img] + [bot] * _MAXR, axis=0)
    left = pimg[:, 0:1]
    right = pimg[:, _W - 1:_W]
    pimg = jnp.concatenate([left] * _MAXR + [pimg] + [right] * _MAXR,
                           axis=1)
    for r in range(1, _MAXR + 1):
        rs = pimg[_MAXR - r:_MAXR - r + _H, :]
        for k in range(-r + 1, r + 1):
            rs = rs + pimg[_MAXR + k:_MAXR + k + _H, :]
        bs = rs[:, _MAXR - r:_MAXR - r + _W]
        for k in range(-r + 1, r + 1):
            bs = bs + rs[:, _MAXR + k:_MAXR + k + _W]
        m = bs * (1.0 / float((2 * r + 1) ** 2))
        base = (r - 1) * _HP
        rowpad = jnp.concatenate(
            [jnp.broadcast_to(m[0:1, :], (_PAD, _W)), m,
             jnp.broadcast_to(m[_H - 1:_H, :], (_PAD, _W))], axis=0)
        out_ref[base:base + _HP, _PAD:_PAD + _W] = rowpad
        lcol = out_ref[base:base + _HP, _PAD:_PAD + 1]
        out_ref[base:base + _HP, 0:_PAD] = jnp.broadcast_to(lcol,
                                                            (_HP, _PAD))
        rcol = out_ref[base:base + _HP, _PAD + _W - 1:_PAD + _W]
        out_ref[base:base + _HP, _PAD + _W:_WP] = jnp.broadcast_to(
            rcol, (_HP, _PAD))


def _boxmean_rows(x):
    B = x.shape[0]
    return pl.pallas_call(
        _boxmean_body,
        grid=(B,),
        in_specs=[pl.BlockSpec((1, 1, _H, _W), lambda b: (b, 0, 0, 0))],
        out_specs=pl.BlockSpec((_MAXR * _HP, _WP), lambda b: (b, 0)),
        out_shape=jax.ShapeDtypeStruct((B * _MAXR * _HP, _WP), jnp.float32),
    )(x)


def _sc_sample_body(mrows, params, thrt, out,
                    par_v, thr_v, slabs, oba, obb, ssem, oa, ob):
    wid = lax.axis_index("s") * 2 + lax.axis_index("c")
    b = wid // 16
    rem = wid - b * 16
    g = rem // 2
    half = rem - g * 2
    obufs = (oba, obb)
    osems = (oa, ob)

    pltpu.sync_copy(params.at[g], par_v)
    pltpu.sync_copy(thrt.at[g], thr_v)
    par_sy1 = par_v[0]
    par_sy2 = par_v[1]
    par_sx1 = par_v[2]
    par_sx2 = par_v[3]
    ridxv = par_v[4]
    thrv = thr_v[...]
    fbase1 = (ridxv * _SLAB + par_sy1) * _WP + par_sx1
    fbase2 = (ridxv * _SLAB + par_sy2) * _WP + par_sx2

    def unit_body(u, _):
        y0 = half * (2 * _YU) + u * _YU

        def slab_copy(rr):
            return pltpu.make_async_copy(
                mrows.at[pl.ds(((b * _MAXR + rr) * _HP + y0) * _WP,
                               _SLAB * _WP)],
                slabs.at[pl.ds(rr * _SLAB * _WP, _SLAB * _WP)], ssem)

        for rr in range(_MAXR):
            slab_copy(rr).start()
        for rr in range(_MAXR):
            slab_copy(rr).wait()

        def out_copy(c, pi):
            return pltpu.make_async_copy(
                obufs[pi],
                out.at[b, pl.ds(y0 + c * _OCH, _OCH), :,
                       pl.ds(g * 16, 16)],
                osems[pi])

        for c in range(_YU // _OCH):
            pi = c % 2
            if c >= 2:
                out_copy(c - 2, pi).wait()
            ob_ref = obufs[pi]
            for yy in range(0, _OCH, 2):
                y = c * _OCH + yy
                ybase1 = fbase1 + y * _WP
                ybase2 = fbase2 + y * _WP

                @plsc.parallel_loop(0, _W, unroll=4,
                                    carry=(ybase1, ybase2))
                def _cols(x, idx):
                    i1, i2 = idx
                    v1 = plsc.load_gather(slabs, [i1])
                    v2 = plsc.load_gather(slabs, [i2])
                    ob_ref[yy, x, :] = v1 - v2 - thrv
                    v1b = plsc.load_gather(slabs, [i1 + _WP])
                    v2b = plsc.load_gather(slabs, [i2 + _WP])
                    ob_ref[yy + 1, x, :] = v1b - v2b - thrv
                    return (i1 + 1, i2 + 1)

            out_copy(c, pi).start()
        nch = _YU // _OCH
        out_copy(nch - 2, (nch - 2) % 2).wait()
        out_copy(nch - 1, (nch - 1) % 2).wait()
        return 0

    lax.fori_loop(0, 2, unit_body, 0)


def _sc_sample(mrows, params, thrt, B):
    mesh = plsc.VectorSubcoreMesh(core_axis_name="c", subcore_axis_name="s")
    f = functools.partial(
        pl.kernel,
        mesh=mesh,
        out_type=jax.ShapeDtypeStruct((B, _H, _W, _NP), jnp.float32),
        scratch_types=[
            pltpu.VMEM((5, 16), jnp.int32),
            pltpu.VMEM((16,), jnp.float32),
            pltpu.VMEM((_MAXR * _SLAB * _WP,), jnp.float32),
            pltpu.VMEM((_OCH, _W, 16), jnp.float32),
            pltpu.VMEM((_OCH, _W, 16), jnp.float32),
            pltpu.SemaphoreType.DMA,
            pltpu.SemaphoreType.DMA,
            pltpu.SemaphoreType.DMA,
        ],
        compiler_params=pltpu.CompilerParams(use_tc_tiling_on_sc=False,
                                             needs_layout_passes=False),
    )(_sc_sample_body)
    return f(mrows, params, thrt)


@jax.jit
def kernel(x, offset_x1, offset_x2, offset_y1, offset_y2, thresholds, radii):
    B = x.shape[0]
    mrows = _boxmean_rows(x)

    s1y = (jnp.floor(offset_y1) + _PAD).astype(jnp.int32)
    s1x = (jnp.floor(offset_x1) + _PAD).astype(jnp.int32)
    s2y = (jnp.floor(offset_y2) + _PAD).astype(jnp.int32)
    s2x = (jnp.floor(offset_x2) + _PAD).astype(jnp.int32)
    ridx = (radii - 1).astype(jnp.int32)

    params = jnp.stack([s1y, s2y, s1x, s2x, ridx],
                       axis=0).reshape(5, _NG, 16).transpose(1, 0, 2)
    thrt = thresholds.astype(jnp.float32).reshape(_NG, 16)

    out4 = _sc_sample(mrows.reshape(-1), params, thrt, B)
    return jnp.transpose(out4, (0, 3, 1, 2))


# speedup vs baseline: 1.0912x; 1.0051x over previous
import functools

import jax
import jax.numpy as jnp
from jax import lax
from jax.experimental import pallas as pl
from jax.experimental.pallas import tpu as pltpu
from jax.experimental.pallas import tpu_sc as plsc

_H = 224
_W = 224
_PAD = 16
_MAXR = 4
_HP = _H + 2 * _PAD
_WP = _W + 2 * _PAD
_NP = 128
_NG = _NP // 16
_YU = 56
_SLAB = _YU + 2 * _PAD
_OCH = 4


def _boxmean_body(x_ref, out_ref):
    img = x_ref[0, 0]
    top = img[0:1, :]
    bot = img[_H - 1:_H, :]
    pimg = jnp.concatenate([top] * _MAXR + [img] + [bot] * _MAXR, axis=0)
    left = pimg[:, 0:1]
    right = pimg[:, _W - 1:_W]
    pimg = jnp.concatenate([left] * _MAXR + [pimg] + [right] * _MAXR,
                           axis=1)
    for r in range(1, _MAXR + 1):
        rs = pimg[_MAXR - r:_MAXR - r + _H, :]
        for k in range(-r + 1, r + 1):
            rs = rs + pimg[_MAXR + k:_MAXR + k + _H, :]
        bs = rs[:, _MAXR - r:_MAXR - r + _W]
        for k in range(-r + 1, r + 1):
            bs = bs + rs[:, _MAXR + k:_MAXR + k + _W]
        m = bs * (1.0 / float((2 * r + 1) ** 2))
        base = (r - 1) * _HP
        rowpad = jnp.concatenate(
            [jnp.broadcast_to(m[0:1, :], (_PAD, _W)), m,
             jnp.broadcast_to(m[_H - 1:_H, :], (_PAD, _W))], axis=0)
        out_ref[base:base + _HP, _PAD:_PAD + _W] = rowpad
        lcol = out_ref[base:base + _HP, _PAD:_PAD + 1]
        out_ref[base:base + _HP, 0:_PAD] = jnp.broadcast_to(lcol,
                                                            (_HP, _PAD))
        rcol = out_ref[base:base + _HP, _PAD + _W - 1:_PAD + _W]
        out_ref[base:base + _HP, _PAD + _W:_WP] = jnp.broadcast_to(
            rcol, (_HP, _PAD))


def _boxmean_rows(x):
    B = x.shape[0]
    return pl.pallas_call(
        _boxmean_body,
        grid=(B,),
        in_specs=[pl.BlockSpec((1, 1, _H, _W), lambda b: (b, 0, 0, 0))],
        out_specs=pl.BlockSpec((_MAXR * _HP, _WP), lambda b: (b, 0)),
        out_shape=jax.ShapeDtypeStruct((B * _MAXR * _HP, _WP), jnp.float32),
    )(x)


def _sc_sample_body(mrows, params, thrt, out,
                    par_v, thr_v, slabs, oba, obb, ssem, oa, ob):
    wid = lax.axis_index("s") * 2 + lax.axis_index("c")
    b = wid // 16
    rem = wid - b * 16
    g = rem // 2
    half = rem - g * 2
    obufs = (oba, obb)
    osems = (oa, ob)

    pltpu.sync_copy(params.at[g], par_v)
    pltpu.sync_copy(thrt.at[g], thr_v)
    par_sy1 = par_v[0]
    par_sy2 = par_v[1]
    par_sx1 = par_v[2]
    par_sx2 = par_v[3]
    ridxv = par_v[4]
    thrv = thr_v[...]
    fbase1 = (ridxv * _SLAB + par_sy1) * _WP + par_sx1
    fbase2 = (ridxv * _SLAB + par_sy2) * _WP + par_sx2

    def unit_body(u, _):
        y0 = half * (2 * _YU) + u * _YU

        def slab_copy(rr):
            return pltpu.make_async_copy(
                mrows.at[pl.ds(((b * _MAXR + rr) * _HP + y0) * _WP,
                               _SLAB * _WP)],
                slabs.at[pl.ds(rr * _SLAB * _WP, _SLAB * _WP)], ssem)

        for rr in range(_MAXR):
            slab_copy(rr).start()
        for rr in range(_MAXR):
            slab_copy(rr).wait()

        def out_copy(c, pi):
            return pltpu.make_async_copy(
                obufs[pi],
                out.at[b, pl.ds(y0 + c * _OCH, _OCH), :,
                       pl.ds(g * 16, 16)],
                osems[pi])

        for c in range(_YU // _OCH):
            pi = c % 2
            if c >= 2:
                out_copy(c - 2, pi).wait()
            ob_ref = obufs[pi]
            y = c * _OCH
            ybase1 = fbase1 + y * _WP
            ybase2 = fbase2 + y * _WP

            @plsc.parallel_loop(0, _W, unroll=2,
                                carry=(ybase1, ybase2))
            def _cols(x, idx):
                i1, i2 = idx
                for yy in range(_OCH):
                    v1 = plsc.load_gather(slabs, [i1 + yy * _WP])
                    v2 = plsc.load_gather(slabs, [i2 + yy * _WP])
                    ob_ref[yy, x, :] = v1 - v2 - thrv
                return (i1 + 1, i2 + 1)

            out_copy(c, pi).start()
        nch = _YU // _OCH
        out_copy(nch - 2, (nch - 2) % 2).wait()
        out_copy(nch - 1, (nch - 1) % 2).wait()
        return 0

    lax.fori_loop(0, 2, unit_body, 0)


def _sc_sample(mrows, params, thrt, B):
    mesh = plsc.VectorSubcoreMesh(core_axis_name="c", subcore_axis_name="s")
    f = functools.partial(
        pl.kernel,
        mesh=mesh,
        out_type=jax.ShapeDtypeStruct((B, _H, _W, _NP), jnp.float32),
        scratch_types=[
            pltpu.VMEM((5, 16), jnp.int32),
            pltpu.VMEM((16,), jnp.float32),
            pltpu.VMEM((_MAXR * _SLAB * _WP,), jnp.float32),
            pltpu.VMEM((_OCH, _W, 16), jnp.float32),
            pltpu.VMEM((_OCH, _W, 16), jnp.float32),
            pltpu.SemaphoreType.DMA,
            pltpu.SemaphoreType.DMA,
            pltpu.SemaphoreType.DMA,
        ],
        compiler_params=pltpu.CompilerParams(use_tc_tiling_on_sc=False,
                                             needs_layout_passes=False),
    )(_sc_sample_body)
    return f(mrows, params, thrt)


@jax.jit
def kernel(x, offset_x1, offset_x2, offset_y1, offset_y2, thresholds, radii):
    B = x.shape[0]
    mrows = _boxmean_rows(x)

    s1y = (jnp.floor(offset_y1) + _PAD).astype(jnp.int32)
    s1x = (jnp.floor(offset_x1) + _PAD).astype(jnp.int32)
    s2y = (jnp.floor(offset_y2) + _PAD).astype(jnp.int32)
    s2x = (jnp.floor(offset_x2) + _PAD).astype(jnp.int32)
    ridx = (radii - 1).astype(jnp.int32)

    params = jnp.stack([s1y, s2y, s1x, s2x, ridx],
                       axis=0).reshape(5, _NG, 16).transpose(1, 0, 2)
    thrt = thresholds.astype(jnp.float32).reshape(_NG, 16)

    out4 = _sc_sample(mrows.reshape(-1), params, thrt, B)
    return jnp.transpose(out4, (0, 3, 1, 2))
